# Initial kernel scaffold; baseline (speedup 1.0000x reference)
import jax
import jax.numpy as jnp
from jax.experimental import pallas as pl


def kernel(latents, codebook):
    raise NotImplementedError("write your pallas kernel here")


# speedup vs baseline: 3.6087x; 3.6087x over previous
import jax
import jax.numpy as jnp
from jax.experimental import pallas as pl
from jax.experimental.pallas import tpu as pltpu

R = 1024
CD = 64
K = 1024
BR = 128


def _vq_body(x_ref, c_ref, out_ref, loss_ref):
    x = x_ref[...]
    c = c_ref[...]
    dot = jax.lax.dot_general(x, c, (((1,), (1,)), ((), ())),
                              preferred_element_type=jnp.float32,
                              precision=jax.lax.Precision.HIGHEST)
    ones = jnp.ones((8, CD), jnp.float32)
    nc8 = jax.lax.dot_general(ones, c * c, (((1,), (1,)), ((), ())),
                              preferred_element_type=jnp.float32,
                              precision=jax.lax.Precision.HIGHEST)
    nc = nc8[0:1, :]
    scores = nc - 2.0 * dot

    iota_k = jax.lax.broadcasted_iota(jnp.int32, (BR, K), 1)

    m1 = jnp.min(scores, axis=1, keepdims=True)
    i1 = jnp.min(jnp.where(scores == m1, iota_k, K), axis=1, keepdims=True)

    masked = jnp.where(iota_k == i1, jnp.inf, scores)
    m2 = jnp.min(masked, axis=1, keepdims=True)
    i2 = jnp.min(jnp.where(masked == m2, iota_k, K), axis=1, keepdims=True)

    oh1 = (iota_k == i1).astype(jnp.float32)
    oh2 = (iota_k == i2).astype(jnp.float32)
    q1 = jax.lax.dot_general(oh1, c, (((1,), (0,)), ((), ())),
                             preferred_element_type=jnp.float32,
                              precision=jax.lax.Precision.HIGHEST)
    q2 = jax.lax.dot_general(oh2, c, (((1,), (0,)), ((), ())),
                             preferred_element_type=jnp.float32,
                              precision=jax.lax.Precision.HIGHEST)

    d1 = jnp.sum((x - q1) ** 2, axis=1, keepdims=True)
    d2 = jnp.sum((x - q2) ** 2, axis=1, keepdims=True)
    s1 = jnp.sqrt(d1)
    s2 = jnp.sqrt(d2)
    use2 = (s2 < s1) | ((s2 == s1) & (i2 < i1))

    q = jnp.where(use2, q2, q1)
    d = jnp.where(use2, d2, d1)
    out_ref[...] = q

    blk = 1.25 * jnp.sum(d) / (R * CD)

    @pl.when(pl.program_id(0) == 0)
    def _init():
        loss_ref[0, 0] = 0.0

    loss_ref[0, 0] += blk


def kernel(latents, codebook):
    x = latents.reshape(R, CD)
    q, loss = pl.pallas_call(
        _vq_body,
        grid=(R // BR,),
        out_shape=(
            jax.ShapeDtypeStruct((R, CD), jnp.float32),
            jax.ShapeDtypeStruct((1, 1), jnp.float32),
        ),
        in_specs=(
            pl.BlockSpec((BR, CD), lambda i: (i, 0)),
            pl.BlockSpec((K, CD), lambda i: (0, 0)),
        ),
        out_specs=(
            pl.BlockSpec((BR, CD), lambda i: (i, 0)),
            pl.BlockSpec(memory_space=pltpu.SMEM),
        ),
    )(x, codebook)
    out = q.reshape(latents.shape[0], R // latents.shape[0] * CD)
    return out, loss[0, 0]
